# baseline (speedup 1.0000x reference)
import functools

import jax
import jax.numpy as jnp
from jax import lax
from jax.experimental import pallas as pl
from jax.experimental.pallas import tpu as pltpu
from jax.experimental.pallas import tpu_sc as plsc

_N = 10000
_D = 128
_E = 320000
_NC = 2
_NS = 16
_NW = _NC * _NS
_CHUNK = 80
_NCHUNK = 128
_EPW = _NCHUNK * _CHUNK
_NACC = _N + 16
_NBUF = 4
_IB = 8
_OWN = 624
_REM = _N - _NS * _OWN
_LANES = 16


def _seg_body(h_hbm, src_hbm, dst_hbm, out_hbm, acc, *sc):
    rows = list(sc[0:_NBUF])
    gsem = list(sc[_NBUF:2 * _NBUF])
    sidx = list(sc[2 * _NBUF:2 * _NBUF + _IB])
    didx = list(sc[2 * _NBUF + _IB:2 * _NBUF + 2 * _IB])
    isem = list(sc[2 * _NBUF + 2 * _IB:2 * _NBUF + 3 * _IB])
    r0 = rows[0]
    cid = lax.axis_index("c")
    sid = lax.axis_index("s")
    wid = sid * _NC + cid
    ebase = wid * _EPW

    def fire_idx(c, k):
        base = pl.multiple_of(ebase + c * _CHUNK, 8)
        pltpu.async_copy(src_hbm.at[pl.ds(base, _CHUNK)], sidx[k], isem[k])
        pltpu.async_copy(dst_hbm.at[pl.ds(base, _CHUNK)], didx[k], isem[k])

    def wait_idx(k):
        pltpu.make_async_copy(src_hbm.at[pl.ds(0, _CHUNK)], sidx[k],
                              isem[k]).wait()
        pltpu.make_async_copy(src_hbm.at[pl.ds(0, _CHUNK)], didx[k],
                              isem[k]).wait()

    def wait_gather(b):
        pltpu.make_async_copy(h_hbm.at[pl.ds(0, _CHUNK)], rows[b],
                              gsem[b]).wait()

    for k in range(_IB):
        fire_idx(jnp.int32(k), k)

    def zrow(i, carry):
        for j in range(_D // _LANES):
            r0[i, pl.ds(j * _LANES, _LANES)] = jnp.zeros((_LANES,), jnp.float32)
        return carry

    lax.fori_loop(0, _CHUNK, zrow, 0)
    rbase = sid * _OWN
    nfull = _OWN // _CHUNK
    tail = _OWN - nfull * _CHUNK
    for j in range(nfull):
        off = pl.multiple_of(rbase + j * _CHUNK, 8)
        pltpu.sync_copy(r0, acc.at[pl.ds(off, _CHUNK)])
    off = pl.multiple_of(rbase + nfull * _CHUNK, 8)
    pltpu.sync_copy(r0.at[pl.ds(0, tail)], acc.at[pl.ds(off, tail)])

    @pl.when(sid == _NS - 1)
    def _zero_rem():
        pltpu.sync_copy(r0.at[pl.ds(0, _REM)],
                        acc.at[pl.ds(_NS * _OWN, _REM)])

    for b in range(_NBUF):
        wait_idx(b)
        pltpu.async_copy(h_hbm.at[sidx[b]], rows[b], gsem[b])
    plsc.subcore_barrier()

    def body(i, carry):
        for k in range(_IB):
            c = i * _IB + k
            b = k % _NBUF
            k4 = (k + _NBUF) % _IB
            wait_gather(b)
            pltpu.sync_copy(rows[b], acc.at[didx[k]], add=True)
            wait_idx(k4)
            pltpu.async_copy(h_hbm.at[sidx[k4]], rows[b], gsem[b])
            fire_idx(c + _IB, k)
        return carry

    lax.fori_loop(0, _NCHUNK // _IB, body, 0)
    for j in range(_NBUF):
        wait_idx((_NCHUNK + _NBUF + j) % _IB)
        wait_gather((_NCHUNK + j) % _NBUF)
    plsc.subcore_barrier()

    obase = cid * _N + rbase
    nfull = _OWN // _CHUNK
    for j in range(nfull + 1):
        w = _CHUNK if j < nfull else tail
        buf = rows[j % _NBUF]
        aoff = pl.multiple_of(rbase + j * _CHUNK, 8)
        ooff = pl.multiple_of(obase + j * _CHUNK, 8)
        pltpu.sync_copy(acc.at[pl.ds(aoff, w)], buf.at[pl.ds(0, w)])
        pltpu.sync_copy(buf.at[pl.ds(0, w)], out_hbm.at[pl.ds(ooff, w)])

    @pl.when(sid == _NS - 1)
    def _wb_rem():
        pltpu.sync_copy(acc.at[pl.ds(_NS * _OWN, _REM)], r0.at[pl.ds(0, _REM)])
        ooff = pl.multiple_of(cid * _N + _NS * _OWN, 8)
        pltpu.sync_copy(r0.at[pl.ds(0, _REM)], out_hbm.at[pl.ds(ooff, _REM)])


_segsum = functools.partial(
    pl.kernel,
    out_type=jax.ShapeDtypeStruct((_NC * _N, _D), jnp.float32),
    mesh=plsc.VectorSubcoreMesh(core_axis_name="c", subcore_axis_name="s"),
    scratch_types=(
        [pltpu.VMEM_SHARED((_NACC, _D), jnp.float32)]
        + [pltpu.VMEM((_CHUNK, _D), jnp.float32)] * _NBUF
        + [pltpu.SemaphoreType.DMA] * _NBUF
        + [pltpu.VMEM((_CHUNK,), jnp.int32)] * _IB
        + [pltpu.VMEM((_CHUNK,), jnp.int32)] * _IB
        + [pltpu.SemaphoreType.DMA] * _IB
    ),
)(_seg_body)


def _mlp1_body(s_ref, x_ref, p_ref, W1_ref, b1_ref, g1_ref, be1_ref, W2_ref,
               b2_ref, o_ref):
    h = x_ref[...] * s_ref[0, 0] + p_ref[0] + p_ref[1]
    z = jnp.dot(h, W1_ref[...], preferred_element_type=jnp.float32) + b1_ref[...]
    mu = jnp.mean(z, axis=0, keepdims=True)
    zc = z - mu
    var = jnp.mean(zc * zc, axis=0, keepdims=True)
    z = zc * lax.rsqrt(var + 1e-5) * g1_ref[...] + be1_ref[...]
    z = jnp.maximum(z, 0.0)
    z = jnp.dot(z, W2_ref[...], preferred_element_type=jnp.float32) + b2_ref[...]
    o_ref[...] = jnp.maximum(z, 0.0)


def _mlp2_body(s_ref, h_ref, p_ref, W3_ref, b3_ref, g2_ref, be2_ref, W4_ref,
               b4_ref, g3_ref, be3_ref, Wfc_ref, bfc_ref, o_ref):
    h = h_ref[...] * s_ref[0, 0] + p_ref[0] + p_ref[1]
    z = jnp.dot(h, W3_ref[...], preferred_element_type=jnp.float32) + b3_ref[...]
    mu = jnp.mean(z, axis=0, keepdims=True)
    zc = z - mu
    var = jnp.mean(zc * zc, axis=0, keepdims=True)
    z = zc * lax.rsqrt(var + 1e-5) * g2_ref[...] + be2_ref[...]
    z = jnp.maximum(z, 0.0)
    z = jnp.dot(z, W4_ref[...], preferred_element_type=jnp.float32) + b4_ref[...]
    mu2 = jnp.mean(z, axis=0, keepdims=True)
    zc2 = z - mu2
    var2 = jnp.mean(zc2 * zc2, axis=0, keepdims=True)
    z = zc2 * lax.rsqrt(var2 + 1e-5) * g3_ref[...] + be3_ref[...]
    z = jnp.maximum(z, 0.0)
    o_ref[...] = (jnp.dot(z, Wfc_ref[...], preferred_element_type=jnp.float32)
                  + bfc_ref[...])


def kernel(x, edge_index, eps1, W1, b1, g1, be1, W2, b2, eps2, W3, b3, g2,
           be2, W4, b4, g3, be3, Wfc, bfc):
    src = edge_index[0]
    dst = edge_index[1]

    npad = _EPW - _E // _NW
    src_p = jnp.concatenate(
        [jnp.concatenate(
            [src.reshape(_NW, _E // _NW),
             jnp.zeros((_NW, npad), jnp.int32)], axis=1).reshape(_NW * _EPW),
         jnp.zeros((_IB * _CHUNK,), jnp.int32)])
    dst_p = jnp.concatenate(
        [jnp.concatenate(
            [dst.reshape(_NW, _E // _NW),
             jnp.full((_NW, npad), _N, jnp.int32)], axis=1).reshape(_NW * _EPW),
         jnp.full((_IB * _CHUNK,), _N, jnp.int32)])

    p1 = _segsum(x, src_p, dst_p).reshape(2, _N, _D)
    h1 = pl.pallas_call(
        _mlp1_body,
        out_shape=jax.ShapeDtypeStruct((_N, _D), jnp.float32),
    )(
        (1.0 + eps1).reshape(1, 1), x, p1, W1, b1.reshape(1, _D),
        g1.reshape(1, _D), be1.reshape(1, _D), W2, b2.reshape(1, _D),
    )

    p2 = _segsum(h1, src_p, dst_p).reshape(2, _N, _D)
    out = pl.pallas_call(
        _mlp2_body,
        out_shape=jax.ShapeDtypeStruct((_N, Wfc.shape[1]), jnp.float32),
    )(
        (1.0 + eps2).reshape(1, 1), h1, p2, W3, b3.reshape(1, _D),
        g2.reshape(1, _D), be2.reshape(1, _D), W4, b4.reshape(1, _D),
        g3.reshape(1, _D), be3.reshape(1, _D), Wfc,
        bfc.reshape(1, bfc.shape[0]),
    )
    return out


# speedup vs baseline: 1.8275x; 1.8275x over previous
import functools

import jax
import jax.numpy as jnp
from jax import lax
from jax.experimental import pallas as pl
from jax.experimental.pallas import tpu as pltpu
from jax.experimental.pallas import tpu_sc as plsc

_N = 10000
_D = 128
_E = 320000
_NC = 2
_NS = 16
_NW = _NC * _NS
_CHUNK = 80
_NCHUNK = 126
_EPW = _NCHUNK * _CHUNK
_NACC = _N + 16
_NBUF = 3
_IB = 6
_OWN = 624
_REM = _N - _NS * _OWN
_LANES = 16


def _seg_body(h_hbm, src_hbm, dst_hbm, out_hbm, acc, *sc):
    rows = list(sc[0:_NBUF])
    gsem = list(sc[_NBUF:2 * _NBUF])
    sidx = list(sc[2 * _NBUF:2 * _NBUF + _IB])
    didx = list(sc[2 * _NBUF + _IB:2 * _NBUF + 2 * _IB])
    isem = list(sc[2 * _NBUF + 2 * _IB:2 * _NBUF + 3 * _IB])
    r0 = rows[0]
    cid = lax.axis_index("c")
    sid = lax.axis_index("s")
    wid = sid * _NC + cid
    ebase = wid * _EPW

    def fire_idx(c, k):
        base = pl.multiple_of(ebase + c * _CHUNK, 8)
        pltpu.async_copy(src_hbm.at[pl.ds(base, _CHUNK)], sidx[k], isem[k])
        pltpu.async_copy(dst_hbm.at[pl.ds(base, _CHUNK)], didx[k], isem[k])

    def wait_idx(k):
        pltpu.make_async_copy(src_hbm.at[pl.ds(0, _CHUNK)], sidx[k],
                              isem[k]).wait()
        pltpu.make_async_copy(src_hbm.at[pl.ds(0, _CHUNK)], didx[k],
                              isem[k]).wait()

    def wait_gather(b):
        pltpu.make_async_copy(h_hbm.at[pl.ds(0, _CHUNK)], rows[b],
                              gsem[b]).wait()

    for k in range(_IB):
        fire_idx(jnp.int32(k), k)

    def zrow(i, carry):
        for j in range(_D // _LANES):
            r0[i, pl.ds(j * _LANES, _LANES)] = jnp.zeros((_LANES,), jnp.float32)
        return carry

    lax.fori_loop(0, _CHUNK, zrow, 0)
    rbase = sid * _OWN
    nfull = _OWN // _CHUNK
    tail = _OWN - nfull * _CHUNK
    for j in range(nfull):
        off = pl.multiple_of(rbase + j * _CHUNK, 8)
        pltpu.sync_copy(r0, acc.at[pl.ds(off, _CHUNK)])
    off = pl.multiple_of(rbase + nfull * _CHUNK, 8)
    pltpu.sync_copy(r0.at[pl.ds(0, tail)], acc.at[pl.ds(off, tail)])

    @pl.when(sid == _NS - 1)
    def _zero_rem():
        pltpu.sync_copy(r0.at[pl.ds(0, _REM)],
                        acc.at[pl.ds(_NS * _OWN, _REM)])

    for b in range(_NBUF):
        wait_idx(b)
        pltpu.async_copy(h_hbm.at[sidx[b]], rows[b], gsem[b])
    plsc.subcore_barrier()

    def body(i, carry):
        for k in range(_IB):
            c = i * _IB + k
            b = k % _NBUF
            k4 = (k + _NBUF) % _IB
            wait_gather(b)
            pltpu.sync_copy(rows[b], acc.at[didx[k]], add=True)
            wait_idx(k4)
            pltpu.async_copy(h_hbm.at[sidx[k4]], rows[b], gsem[b])
            fire_idx(c + _IB, k)
        return carry

    lax.fori_loop(0, _NCHUNK // _IB, body, 0)
    for j in range(_NBUF):
        wait_idx((_NCHUNK + _NBUF + j) % _IB)
        wait_gather((_NCHUNK + j) % _NBUF)
    plsc.subcore_barrier()

    obase = cid * _N + rbase
    nfull = _OWN // _CHUNK
    for j in range(nfull + 1):
        w = _CHUNK if j < nfull else tail
        buf = rows[j % _NBUF]
        aoff = pl.multiple_of(rbase + j * _CHUNK, 8)
        ooff = pl.multiple_of(obase + j * _CHUNK, 8)
        pltpu.sync_copy(acc.at[pl.ds(aoff, w)], buf.at[pl.ds(0, w)])
        pltpu.sync_copy(buf.at[pl.ds(0, w)], out_hbm.at[pl.ds(ooff, w)])

    @pl.when(sid == _NS - 1)
    def _wb_rem():
        pltpu.sync_copy(acc.at[pl.ds(_NS * _OWN, _REM)], r0.at[pl.ds(0, _REM)])
        ooff = pl.multiple_of(cid * _N + _NS * _OWN, 8)
        pltpu.sync_copy(r0.at[pl.ds(0, _REM)], out_hbm.at[pl.ds(ooff, _REM)])


_segsum = functools.partial(
    pl.kernel,
    out_type=jax.ShapeDtypeStruct((_NC * _N, _D), jnp.float32),
    mesh=plsc.VectorSubcoreMesh(core_axis_name="c", subcore_axis_name="s"),
    scratch_types=(
        [pltpu.VMEM_SHARED((_NACC, _D), jnp.float32)]
        + [pltpu.VMEM((_CHUNK, _D), jnp.float32)] * _NBUF
        + [pltpu.SemaphoreType.DMA] * _NBUF
        + [pltpu.VMEM((_CHUNK,), jnp.int32)] * _IB
        + [pltpu.VMEM((_CHUNK,), jnp.int32)] * _IB
        + [pltpu.SemaphoreType.DMA] * _IB
    ),
)(_seg_body)


def _mlp1_body(s_ref, x_ref, p_ref, W1_ref, b1_ref, g1_ref, be1_ref, W2_ref,
               b2_ref, o_ref):
    h = x_ref[...] * s_ref[0, 0] + p_ref[0] + p_ref[1]
    z = jnp.dot(h, W1_ref[...], preferred_element_type=jnp.float32) + b1_ref[...]
    mu = jnp.mean(z, axis=0, keepdims=True)
    zc = z - mu
    var = jnp.mean(zc * zc, axis=0, keepdims=True)
    z = zc * lax.rsqrt(var + 1e-5) * g1_ref[...] + be1_ref[...]
    z = jnp.maximum(z, 0.0)
    z = jnp.dot(z, W2_ref[...], preferred_element_type=jnp.float32) + b2_ref[...]
    o_ref[...] = jnp.maximum(z, 0.0)


def _mlp2_body(s_ref, h_ref, p_ref, W3_ref, b3_ref, g2_ref, be2_ref, W4_ref,
               b4_ref, g3_ref, be3_ref, Wfc_ref, bfc_ref, o_ref):
    h = h_ref[...] * s_ref[0, 0] + p_ref[0] + p_ref[1]
    z = jnp.dot(h, W3_ref[...], preferred_element_type=jnp.float32) + b3_ref[...]
    mu = jnp.mean(z, axis=0, keepdims=True)
    zc = z - mu
    var = jnp.mean(zc * zc, axis=0, keepdims=True)
    z = zc * lax.rsqrt(var + 1e-5) * g2_ref[...] + be2_ref[...]
    z = jnp.maximum(z, 0.0)
    z = jnp.dot(z, W4_ref[...], preferred_element_type=jnp.float32) + b4_ref[...]
    mu2 = jnp.mean(z, axis=0, keepdims=True)
    zc2 = z - mu2
    var2 = jnp.mean(zc2 * zc2, axis=0, keepdims=True)
    z = zc2 * lax.rsqrt(var2 + 1e-5) * g3_ref[...] + be3_ref[...]
    z = jnp.maximum(z, 0.0)
    o_ref[...] = (jnp.dot(z, Wfc_ref[...], preferred_element_type=jnp.float32)
                  + bfc_ref[...])


def kernel(x, edge_index, eps1, W1, b1, g1, be1, W2, b2, eps2, W3, b3, g2,
           be2, W4, b4, g3, be3, Wfc, bfc):
    src = edge_index[0]
    dst = edge_index[1]

    npad = _EPW - _E // _NW
    src_p = jnp.concatenate(
        [jnp.concatenate(
            [src.reshape(_NW, _E // _NW),
             jnp.zeros((_NW, npad), jnp.int32)], axis=1).reshape(_NW * _EPW),
         jnp.zeros((_IB * _CHUNK,), jnp.int32)])
    dst_p = jnp.concatenate(
        [jnp.concatenate(
            [dst.reshape(_NW, _E // _NW),
             jnp.full((_NW, npad), _N, jnp.int32)], axis=1).reshape(_NW * _EPW),
         jnp.full((_IB * _CHUNK,), _N, jnp.int32)])

    p1 = _segsum(x, src_p, dst_p).reshape(2, _N, _D)
    h1 = pl.pallas_call(
        _mlp1_body,
        out_shape=jax.ShapeDtypeStruct((_N, _D), jnp.float32),
    )(
        (1.0 + eps1).reshape(1, 1), x, p1, W1, b1.reshape(1, _D),
        g1.reshape(1, _D), be1.reshape(1, _D), W2, b2.reshape(1, _D),
    )

    p2 = _segsum(h1, src_p, dst_p).reshape(2, _N, _D)
    out = pl.pallas_call(
        _mlp2_body,
        out_shape=jax.ShapeDtypeStruct((_N, Wfc.shape[1]), jnp.float32),
    )(
        (1.0 + eps2).reshape(1, 1), h1, p2, W3, b3.reshape(1, _D),
        g2.reshape(1, _D), be2.reshape(1, _D), W4, b4.reshape(1, _D),
        g3.reshape(1, _D), be3.reshape(1, _D), Wfc,
        bfc.reshape(1, bfc.shape[0]),
    )
    return out


# speedup vs baseline: 2.4114x; 1.3195x over previous
import functools

import jax
import jax.numpy as jnp
from jax import lax
from jax.experimental import pallas as pl
from jax.experimental.pallas import tpu as pltpu
from jax.experimental.pallas import tpu_sc as plsc

_N = 10000
_D = 128
_E = 320000
_NC = 2
_NS = 16
_NW = _NC * _NS
_CHUNK = 88
_NCHUNK = 114
_EPW = _NCHUNK * _CHUNK
_NACC = _N + 16
_NBUF = 3
_IB = 6
_OWN = 624
_REM = _N - _NS * _OWN
_LANES = 16


def _seg_body(h_hbm, src_hbm, dst_hbm, out_hbm, acc, *sc):
    rows = list(sc[0:_NBUF])
    gsem = list(sc[_NBUF:2 * _NBUF])
    sidx = list(sc[2 * _NBUF:2 * _NBUF + _IB])
    didx = list(sc[2 * _NBUF + _IB:2 * _NBUF + 2 * _IB])
    isem = list(sc[2 * _NBUF + 2 * _IB:2 * _NBUF + 3 * _IB])
    r0 = rows[0]
    cid = lax.axis_index("c")
    sid = lax.axis_index("s")
    wid = sid * _NC + cid
    ebase = wid * _EPW

    def fire_idx(c, k):
        base = pl.multiple_of(ebase + c * _CHUNK, 8)
        pltpu.async_copy(src_hbm.at[pl.ds(base, _CHUNK)], sidx[k], isem[k])
        pltpu.async_copy(dst_hbm.at[pl.ds(base, _CHUNK)], didx[k], isem[k])

    def wait_idx(k):
        pltpu.make_async_copy(src_hbm.at[pl.ds(0, _CHUNK)], sidx[k],
                              isem[k]).wait()
        pltpu.make_async_copy(src_hbm.at[pl.ds(0, _CHUNK)], didx[k],
                              isem[k]).wait()

    def wait_gather(b):
        pltpu.make_async_copy(h_hbm.at[pl.ds(0, _CHUNK)], rows[b],
                              gsem[b]).wait()

    for k in range(_IB):
        fire_idx(jnp.int32(k), k)

    def zrow(i, carry):
        for j in range(_D // _LANES):
            r0[i, pl.ds(j * _LANES, _LANES)] = jnp.zeros((_LANES,), jnp.float32)
        return carry

    lax.fori_loop(0, _CHUNK, zrow, 0)
    rbase = sid * _OWN
    nfull = _OWN // _CHUNK
    tail = _OWN - nfull * _CHUNK
    for j in range(nfull):
        off = pl.multiple_of(rbase + j * _CHUNK, 8)
        pltpu.sync_copy(r0, acc.at[pl.ds(off, _CHUNK)])
    off = pl.multiple_of(rbase + nfull * _CHUNK, 8)
    pltpu.sync_copy(r0.at[pl.ds(0, tail)], acc.at[pl.ds(off, tail)])

    @pl.when(sid == _NS - 1)
    def _zero_rem():
        pltpu.sync_copy(r0.at[pl.ds(0, _REM)],
                        acc.at[pl.ds(_NS * _OWN, _REM)])

    for b in range(_NBUF):
        wait_idx(b)
        pltpu.async_copy(h_hbm.at[sidx[b]], rows[b], gsem[b])
    plsc.subcore_barrier()

    def body(i, carry):
        for k in range(_IB):
            c = i * _IB + k
            b = k % _NBUF
            k4 = (k + _NBUF) % _IB
            wait_gather(b)
            pltpu.sync_copy(rows[b], acc.at[didx[k]], add=True)
            wait_idx(k4)
            pltpu.async_copy(h_hbm.at[sidx[k4]], rows[b], gsem[b])
            fire_idx(c + _IB, k)
        return carry

    lax.fori_loop(0, _NCHUNK // _IB, body, 0)
    for j in range(_NBUF):
        wait_idx((_NCHUNK + _NBUF + j) % _IB)
        wait_gather((_NCHUNK + j) % _NBUF)
    plsc.subcore_barrier()

    obase = cid * _N + rbase
    nfull = _OWN // _CHUNK
    for j in range(nfull + 1):
        w = _CHUNK if j < nfull else tail
        buf = rows[j % _NBUF]
        aoff = pl.multiple_of(rbase + j * _CHUNK, 8)
        ooff = pl.multiple_of(obase + j * _CHUNK, 8)
        pltpu.sync_copy(acc.at[pl.ds(aoff, w)], buf.at[pl.ds(0, w)])
        pltpu.sync_copy(buf.at[pl.ds(0, w)], out_hbm.at[pl.ds(ooff, w)])

    @pl.when(sid == _NS - 1)
    def _wb_rem():
        pltpu.sync_copy(acc.at[pl.ds(_NS * _OWN, _REM)], r0.at[pl.ds(0, _REM)])
        ooff = pl.multiple_of(cid * _N + _NS * _OWN, 8)
        pltpu.sync_copy(r0.at[pl.ds(0, _REM)], out_hbm.at[pl.ds(ooff, _REM)])


_segsum = functools.partial(
    pl.kernel,
    out_type=jax.ShapeDtypeStruct((_NC * _N, _D), jnp.float32),
    mesh=plsc.VectorSubcoreMesh(core_axis_name="c", subcore_axis_name="s"),
    scratch_types=(
        [pltpu.VMEM_SHARED((_NACC, _D), jnp.float32)]
        + [pltpu.VMEM((_CHUNK, _D), jnp.float32)] * _NBUF
        + [pltpu.SemaphoreType.DMA] * _NBUF
        + [pltpu.VMEM((_CHUNK,), jnp.int32)] * _IB
        + [pltpu.VMEM((_CHUNK,), jnp.int32)] * _IB
        + [pltpu.SemaphoreType.DMA] * _IB
    ),
)(_seg_body)


def _mlp1_body(s_ref, x_ref, p_ref, W1_ref, b1_ref, g1_ref, be1_ref, W2_ref,
               b2_ref, o_ref):
    h = x_ref[...] * s_ref[0, 0] + p_ref[0] + p_ref[1]
    z = jnp.dot(h, W1_ref[...], preferred_element_type=jnp.float32) + b1_ref[...]
    mu = jnp.mean(z, axis=0, keepdims=True)
    zc = z - mu
    var = jnp.mean(zc * zc, axis=0, keepdims=True)
    z = zc * lax.rsqrt(var + 1e-5) * g1_ref[...] + be1_ref[...]
    z = jnp.maximum(z, 0.0)
    z = jnp.dot(z, W2_ref[...], preferred_element_type=jnp.float32) + b2_ref[...]
    o_ref[...] = jnp.maximum(z, 0.0)


def _mlp2_body(s_ref, h_ref, p_ref, W3_ref, b3_ref, g2_ref, be2_ref, W4_ref,
               b4_ref, g3_ref, be3_ref, Wfc_ref, bfc_ref, o_ref):
    h = h_ref[...] * s_ref[0, 0] + p_ref[0] + p_ref[1]
    z = jnp.dot(h, W3_ref[...], preferred_element_type=jnp.float32) + b3_ref[...]
    mu = jnp.mean(z, axis=0, keepdims=True)
    zc = z - mu
    var = jnp.mean(zc * zc, axis=0, keepdims=True)
    z = zc * lax.rsqrt(var + 1e-5) * g2_ref[...] + be2_ref[...]
    z = jnp.maximum(z, 0.0)
    z = jnp.dot(z, W4_ref[...], preferred_element_type=jnp.float32) + b4_ref[...]
    mu2 = jnp.mean(z, axis=0, keepdims=True)
    zc2 = z - mu2
    var2 = jnp.mean(zc2 * zc2, axis=0, keepdims=True)
    z = zc2 * lax.rsqrt(var2 + 1e-5) * g3_ref[...] + be3_ref[...]
    z = jnp.maximum(z, 0.0)
    o_ref[...] = (jnp.dot(z, Wfc_ref[...], preferred_element_type=jnp.float32)
                  + bfc_ref[...])


def kernel(x, edge_index, eps1, W1, b1, g1, be1, W2, b2, eps2, W3, b3, g2,
           be2, W4, b4, g3, be3, Wfc, bfc):
    src = edge_index[0]
    dst = edge_index[1]

    npad = _EPW - _E // _NW
    src_p = jnp.concatenate(
        [jnp.concatenate(
            [src.reshape(_NW, _E // _NW),
             jnp.zeros((_NW, npad), jnp.int32)], axis=1).reshape(_NW * _EPW),
         jnp.zeros((_IB * _CHUNK,), jnp.int32)])
    dst_p = jnp.concatenate(
        [jnp.concatenate(
            [dst.reshape(_NW, _E // _NW),
             jnp.full((_NW, npad), _N, jnp.int32)], axis=1).reshape(_NW * _EPW),
         jnp.full((_IB * _CHUNK,), _N, jnp.int32)])

    p1 = _segsum(x, src_p, dst_p).reshape(2, _N, _D)
    h1 = pl.pallas_call(
        _mlp1_body,
        out_shape=jax.ShapeDtypeStruct((_N, _D), jnp.float32),
    )(
        (1.0 + eps1).reshape(1, 1), x, p1, W1, b1.reshape(1, _D),
        g1.reshape(1, _D), be1.reshape(1, _D), W2, b2.reshape(1, _D),
    )

    p2 = _segsum(h1, src_p, dst_p).reshape(2, _N, _D)
    out = pl.pallas_call(
        _mlp2_body,
        out_shape=jax.ShapeDtypeStruct((_N, Wfc.shape[1]), jnp.float32),
    )(
        (1.0 + eps2).reshape(1, 1), h1, p2, W3, b3.reshape(1, _D),
        g2.reshape(1, _D), be2.reshape(1, _D), W4, b4.reshape(1, _D),
        g3.reshape(1, _D), be3.reshape(1, _D), Wfc,
        bfc.reshape(1, bfc.shape[0]),
    )
    return out
